# baseline (speedup 1.0000x reference)
import functools

import jax
import jax.numpy as jnp
from jax import lax
from jax.experimental import pallas as pl
from jax.experimental.pallas import tpu as pltpu
from jax.experimental.pallas import tpu_sc as plsc

GRID_SIZE = 0.05
SIGMA = 1.0
POINT_INFLUENCE = GRID_SIZE * SIGMA
BN_EPS = 1e-5
NEG_SLOPE = 0.2

_NC, _NS = 2, 16
_NW = _NC * _NS
_CHUNK = 400
_IDXP = 128
_BLK = 200
_BLK_BN = 2000


def _sc_gather(table, posx, posy, posz, idx):
    E = idx.shape[0]
    V, D = table.shape
    per_w = E // _NW
    iters = per_w // _CHUNK
    mesh = plsc.VectorSubcoreMesh(core_axis_name="c", subcore_axis_name="s")

    @functools.partial(
        pl.kernel,
        mesh=mesh,
        compiler_params=pltpu.CompilerParams(needs_layout_passes=False),
        out_type=[
            jax.ShapeDtypeStruct((E, D), jnp.float32),
            jax.ShapeDtypeStruct((3 * E,), jnp.float32),
        ],
        scratch_types=[
            pltpu.VMEM((per_w,), jnp.int32),
            pltpu.VMEM((_CHUNK, D), jnp.float32),
            pltpu.VMEM((V,), jnp.float32),
            pltpu.VMEM((V,), jnp.float32),
            pltpu.VMEM((V,), jnp.float32),
            pltpu.VMEM((3 * _CHUNK,), jnp.float32),
            pltpu.SemaphoreType.DMA,
        ],
    )
    def gath(table_hbm, posx_hbm, posy_hbm, posz_hbm, idx_hbm,
             nx_hbm, np3_hbm,
             idx_v, rows_v, px_v, py_v, pz_v, b3_v, sem):
        wid = lax.axis_index("s") * _NC + lax.axis_index("c")
        base = wid * per_w
        pltpu.sync_copy(idx_hbm.at[pl.ds(base, per_w)], idx_v)
        pltpu.sync_copy(posx_hbm, px_v)
        pltpu.sync_copy(posy_hbm, py_v)
        pltpu.sync_copy(posz_hbm, pz_v)

        def body(i, carry):
            off = base + i * _CHUNK
            loc = i * _CHUNK
            cps = []
            for o in range(0, _CHUNK, _IDXP):
                sz = min(_IDXP, _CHUNK - o)
                cps.append(pltpu.async_copy(
                    table_hbm.at[idx_v.at[pl.ds(loc + o, sz)]],
                    rows_v.at[pl.ds(o, sz)], sem))
            for o in range(0, _CHUNK, 16):
                nbr = idx_v[pl.ds(loc + o, 16)]
                for c, src in ((0, px_v), (1, py_v), (2, pz_v)):
                    b3_v[pl.ds(c * _CHUNK + o, 16)] = (
                        plsc.load_gather(src, [nbr]))
            pltpu.sync_copy(b3_v, np3_hbm.at[pl.ds(off * 3, 3 * _CHUNK)])
            for cp in cps:
                cp.wait()
            pltpu.sync_copy(rows_v, nx_hbm.at[pl.ds(off, _CHUNK)])
            return carry

        lax.fori_loop(0, iters, body, 0)

    return gath(table, posx, posy, posz, idx)


def _conv_body(nx_ref, npx_ref, npy_ref, npz_ref, pos_ref, kp_ref, w_ref,
               out_ref, s1_ref, s2_ref):
    b, kk = npx_ref.shape
    nkp = kp_ref.shape[1]
    dxc = npx_ref[...] - pos_ref[:, 0:1]
    dyc = npy_ref[...] - pos_ref[:, 1:2]
    dzc = npz_ref[...] - pos_ref[:, 2:3]
    dx3 = dxc.reshape(b, 1, kk)
    dy3 = dyc.reshape(b, 1, kk)
    dz3 = dzc.reshape(b, 1, kk)
    kx = kp_ref[0:1, :].reshape(1, nkp, 1)
    ky = kp_ref[1:2, :].reshape(1, nkp, 1)
    kz = kp_ref[2:3, :].reshape(1, nkp, 1)
    d2 = (dx3 - kx) ** 2 + (dy3 - ky) ** 2 + (dz3 - kz) ** 2
    dist = jnp.sqrt(jnp.maximum(d2, 1e-12))
    infl3 = jnp.maximum(0.0, 1.0 - dist / POINT_INFLUENCE)
    weighted = lax.dot_general(
        infl3, nx_ref[...],
        dimension_numbers=(((2,), (1,)), ((0,), (0,))),
        precision=lax.Precision.DEFAULT,
        preferred_element_type=jnp.float32)
    wcat = jnp.concatenate([weighted[:, p, :] for p in range(nkp)], axis=1)
    acc = lax.dot(wcat, w_ref[...],
                  precision=lax.Precision.DEFAULT,
                  preferred_element_type=jnp.float32)
    out_ref[...] = acc

    @pl.when(pl.program_id(0) == 0)
    def _():
        s1_ref[...] = jnp.zeros_like(s1_ref)
        s2_ref[...] = jnp.zeros_like(s2_ref)

    s1_ref[...] += jnp.sum(acc, axis=0, keepdims=True)
    s2_ref[...] += jnp.sum(acc * acc, axis=0, keepdims=True)


def _bn_body(n_rows, o_ref, s1_ref, s2_ref, g_ref, b_ref, y_ref):
    inv_n = 1.0 / float(n_rows)
    mean = s1_ref[...] * inv_n
    var = s2_ref[...] * inv_n - mean * mean
    scale = g_ref[...] * lax.rsqrt(var + BN_EPS)
    y = (o_ref[...] - mean) * scale + b_ref[...]
    y_ref[...] = jnp.where(y >= 0.0, y, NEG_SLOPE * y)


def kernel(pos, x, idx_neighboors, kernel_pts, W, gamma, beta):
    n, d_in = x.shape
    k = idx_neighboors.shape[1]
    nkp, _, d_out = W.shape

    idx_nm = idx_neighboors.astype(jnp.int32).reshape(-1)
    pos_c = jnp.asarray(pos.T, jnp.float32)
    nx, np3 = _sc_gather(x, pos_c[0], pos_c[1], pos_c[2], idx_nm)
    nx3 = nx.reshape(n, k, d_in)
    np3r = np3.reshape((n * k) // _CHUNK, 3, _CHUNK)
    npx2 = np3r[:, 0, :].reshape(n, k)
    npy2 = np3r[:, 1, :].reshape(n, k)
    npz2 = np3r[:, 2, :].reshape(n, k)
    w2 = W.reshape(nkp * d_in, d_out)

    out_raw, s1, s2 = pl.pallas_call(
        _conv_body,
        grid=(n // _BLK,),
        in_specs=[
            pl.BlockSpec((_BLK, k, d_in), lambda i: (i, 0, 0)),
            pl.BlockSpec((_BLK, k), lambda i: (i, 0)),
            pl.BlockSpec((_BLK, k), lambda i: (i, 0)),
            pl.BlockSpec((_BLK, k), lambda i: (i, 0)),
            pl.BlockSpec((_BLK, 3), lambda i: (i, 0)),
            pl.BlockSpec((3, nkp), lambda i: (0, 0)),
            pl.BlockSpec((nkp * d_in, d_out), lambda i: (0, 0)),
        ],
        out_specs=[
            pl.BlockSpec((_BLK, d_out), lambda i: (i, 0)),
            pl.BlockSpec((1, d_out), lambda i: (0, 0)),
            pl.BlockSpec((1, d_out), lambda i: (0, 0)),
        ],
        out_shape=[
            jax.ShapeDtypeStruct((n, d_out), jnp.float32),
            jax.ShapeDtypeStruct((1, d_out), jnp.float32),
            jax.ShapeDtypeStruct((1, d_out), jnp.float32),
        ],
    )(nx3, npx2, npy2, npz2, pos, jnp.asarray(kernel_pts.T, jnp.float32), w2)

    out = pl.pallas_call(
        functools.partial(_bn_body, n),
        grid=(n // _BLK_BN,),
        in_specs=[
            pl.BlockSpec((_BLK_BN, d_out), lambda i: (i, 0)),
            pl.BlockSpec((1, d_out), lambda i: (0, 0)),
            pl.BlockSpec((1, d_out), lambda i: (0, 0)),
            pl.BlockSpec((1, d_out), lambda i: (0, 0)),
            pl.BlockSpec((1, d_out), lambda i: (0, 0)),
        ],
        out_specs=pl.BlockSpec((_BLK_BN, d_out), lambda i: (i, 0)),
        out_shape=jax.ShapeDtypeStruct((n, d_out), jnp.float32),
    )(out_raw, s1, s2, gamma.reshape(1, d_out), beta.reshape(1, d_out))

    return out


# speedup vs baseline: 1.0206x; 1.0206x over previous
import functools

import jax
import jax.numpy as jnp
from jax import lax
from jax.experimental import pallas as pl
from jax.experimental.pallas import tpu as pltpu
from jax.experimental.pallas import tpu_sc as plsc

GRID_SIZE = 0.05
SIGMA = 1.0
POINT_INFLUENCE = GRID_SIZE * SIGMA
BN_EPS = 1e-5
NEG_SLOPE = 0.2

_NC, _NS = 2, 16
_NW = _NC * _NS
_CHUNK = 400
_IDXP = 128
_BLK = 200
_BLK_BN = 2000


def _sc_gather(table, posx, posy, posz, idx):
    E = idx.shape[0]
    V, D = table.shape
    per_w = E // _NW
    iters = per_w // _CHUNK
    mesh = plsc.VectorSubcoreMesh(core_axis_name="c", subcore_axis_name="s")

    @functools.partial(
        pl.kernel,
        mesh=mesh,
        compiler_params=pltpu.CompilerParams(needs_layout_passes=False),
        out_type=[
            jax.ShapeDtypeStruct((E, D), jnp.float32),
            jax.ShapeDtypeStruct((3 * E,), jnp.float32),
        ],
        scratch_types=[
            pltpu.VMEM((per_w,), jnp.int32),
            pltpu.VMEM((_CHUNK, D), jnp.float32),
            pltpu.VMEM((V,), jnp.float32),
            pltpu.VMEM((V,), jnp.float32),
            pltpu.VMEM((V,), jnp.float32),
            pltpu.VMEM((3 * _CHUNK,), jnp.float32),
            pltpu.SemaphoreType.DMA,
        ],
    )
    def gath(table_hbm, posx_hbm, posy_hbm, posz_hbm, idx_hbm,
             nx_hbm, np3_hbm,
             idx_v, rows_v, px_v, py_v, pz_v, b3_v, sem):
        wid = lax.axis_index("s") * _NC + lax.axis_index("c")
        base = wid * per_w
        pltpu.sync_copy(idx_hbm.at[pl.ds(base, per_w)], idx_v)
        pltpu.sync_copy(posx_hbm, px_v)
        pltpu.sync_copy(posy_hbm, py_v)
        pltpu.sync_copy(posz_hbm, pz_v)

        def body(i, carry):
            off = base + i * _CHUNK
            loc = i * _CHUNK
            cps = []
            for o in range(0, _CHUNK, _IDXP):
                sz = min(_IDXP, _CHUNK - o)
                cps.append(pltpu.async_copy(
                    table_hbm.at[idx_v.at[pl.ds(loc + o, sz)]],
                    rows_v.at[pl.ds(o, sz)], sem))
            for o in range(0, _CHUNK, 16):
                nbr = idx_v[pl.ds(loc + o, 16)]
                for c, src in ((0, px_v), (1, py_v), (2, pz_v)):
                    b3_v[pl.ds(c * _CHUNK + o, 16)] = (
                        plsc.load_gather(src, [nbr]))
            pltpu.sync_copy(b3_v, np3_hbm.at[pl.ds(off * 3, 3 * _CHUNK)])
            for cp in cps:
                cp.wait()
            pltpu.sync_copy(rows_v, nx_hbm.at[pl.ds(off, _CHUNK)])
            return carry

        lax.fori_loop(0, iters, body, 0)

    return gath(table, posx, posy, posz, idx)


def _conv_body(nx_ref, npx_ref, npy_ref, npz_ref, pos_ref, kp_ref, w_ref,
               out_ref, s1_ref, s2_ref):
    b, kk = npx_ref.shape
    nkp = kp_ref.shape[1]
    dxc = npx_ref[...] - pos_ref[:, 0:1]
    dyc = npy_ref[...] - pos_ref[:, 1:2]
    dzc = npz_ref[...] - pos_ref[:, 2:3]
    dx3 = dxc.reshape(b, 1, kk)
    dy3 = dyc.reshape(b, 1, kk)
    dz3 = dzc.reshape(b, 1, kk)
    kx = kp_ref[0:1, :].reshape(1, nkp, 1)
    ky = kp_ref[1:2, :].reshape(1, nkp, 1)
    kz = kp_ref[2:3, :].reshape(1, nkp, 1)
    d2 = (dx3 - kx) ** 2 + (dy3 - ky) ** 2 + (dz3 - kz) ** 2
    dist = jnp.sqrt(jnp.maximum(d2, 1e-12))
    infl3 = jnp.maximum(0.0, 1.0 - dist / POINT_INFLUENCE)
    weighted = lax.dot_general(
        infl3, nx_ref[...],
        dimension_numbers=(((2,), (1,)), ((0,), (0,))),
        precision=lax.Precision.DEFAULT,
        preferred_element_type=jnp.float32)
    wcat = jnp.concatenate([weighted[:, p, :] for p in range(nkp)], axis=1)
    acc = lax.dot(wcat, w_ref[...],
                  precision=lax.Precision.DEFAULT,
                  preferred_element_type=jnp.float32)
    out_ref[...] = acc

    @pl.when(pl.program_id(0) == 0)
    def _():
        s1_ref[...] = jnp.zeros_like(s1_ref)
        s2_ref[...] = jnp.zeros_like(s2_ref)

    s1_ref[...] += jnp.sum(acc, axis=0, keepdims=True)
    s2_ref[...] += jnp.sum(acc * acc, axis=0, keepdims=True)


def _bn_body(n_rows, o_ref, s1_ref, s2_ref, g_ref, b_ref, y_ref):
    inv_n = 1.0 / float(n_rows)
    mean = s1_ref[...] * inv_n
    var = s2_ref[...] * inv_n - mean * mean
    scale = g_ref[...] * lax.rsqrt(var + BN_EPS)
    y = (o_ref[...] - mean) * scale + b_ref[...]
    y_ref[...] = jnp.where(y >= 0.0, y, NEG_SLOPE * y)


def kernel(pos, x, idx_neighboors, kernel_pts, W, gamma, beta):
    n, d_in = x.shape
    k = idx_neighboors.shape[1]
    nkp, _, d_out = W.shape

    idx_nm = idx_neighboors.astype(jnp.int32).reshape(-1)
    pos_c = jnp.asarray(pos.T, jnp.float32)
    kp_t = jnp.asarray(kernel_pts.T, jnp.float32)
    w2 = W.reshape(nkp * d_in, d_out)

    outs, stats = [], []
    for r0, r1 in ((0, 6400), (6400, n)):
        rows = r1 - r0
        nx_s, np3_s = _sc_gather(
            x, pos_c[0], pos_c[1], pos_c[2],
            lax.slice_in_dim(idx_nm, r0 * k, r1 * k))
        nx3 = nx_s.reshape(rows, k, d_in)
        np3r = np3_s.reshape((rows * k) // _CHUNK, 3, _CHUNK)
        npx2 = np3r[:, 0, :].reshape(rows, k)
        npy2 = np3r[:, 1, :].reshape(rows, k)
        npz2 = np3r[:, 2, :].reshape(rows, k)
        o_s, s1_s, s2_s = pl.pallas_call(
            _conv_body,
            grid=(rows // _BLK,),
            in_specs=[
                pl.BlockSpec((_BLK, k, d_in), lambda i: (i, 0, 0)),
                pl.BlockSpec((_BLK, k), lambda i: (i, 0)),
                pl.BlockSpec((_BLK, k), lambda i: (i, 0)),
                pl.BlockSpec((_BLK, k), lambda i: (i, 0)),
                pl.BlockSpec((_BLK, 3), lambda i: (i, 0)),
                pl.BlockSpec((3, nkp), lambda i: (0, 0)),
                pl.BlockSpec((nkp * d_in, d_out), lambda i: (0, 0)),
            ],
            out_specs=[
                pl.BlockSpec((_BLK, d_out), lambda i: (i, 0)),
                pl.BlockSpec((1, d_out), lambda i: (0, 0)),
                pl.BlockSpec((1, d_out), lambda i: (0, 0)),
            ],
            out_shape=[
                jax.ShapeDtypeStruct((rows, d_out), jnp.float32),
                jax.ShapeDtypeStruct((1, d_out), jnp.float32),
                jax.ShapeDtypeStruct((1, d_out), jnp.float32),
            ],
        )(nx3, npx2, npy2, npz2, pos[r0:r1], kp_t, w2)
        outs.append(o_s)
        stats.append((s1_s, s2_s))
    out_raw = jnp.concatenate(outs, axis=0)
    s1 = stats[0][0] + stats[1][0]
    s2 = stats[0][1] + stats[1][1]

    out = pl.pallas_call(
        functools.partial(_bn_body, n),
        grid=(n // _BLK_BN,),
        in_specs=[
            pl.BlockSpec((_BLK_BN, d_out), lambda i: (i, 0)),
            pl.BlockSpec((1, d_out), lambda i: (0, 0)),
            pl.BlockSpec((1, d_out), lambda i: (0, 0)),
            pl.BlockSpec((1, d_out), lambda i: (0, 0)),
            pl.BlockSpec((1, d_out), lambda i: (0, 0)),
        ],
        out_specs=pl.BlockSpec((_BLK_BN, d_out), lambda i: (i, 0)),
        out_shape=jax.ShapeDtypeStruct((n, d_out), jnp.float32),
    )(out_raw, s1, s2, gamma.reshape(1, d_out), beta.reshape(1, d_out))

    return out
